# Initial kernel scaffold; baseline (speedup 1.0000x reference)
import jax
import jax.numpy as jnp
from jax.experimental import pallas as pl


def kernel(hidden_states, Wq_idx, bq_idx, Wk_idx, bk_idx, indexer_w, Wq, Wk, Wv, Wo, k):
    raise NotImplementedError("write your pallas kernel here")


# speedup vs baseline: 10.6867x; 10.6867x over previous
import functools

import jax
import jax.numpy as jnp
import numpy as np
from jax import lax
from jax.experimental import pallas as pl
from jax.experimental.pallas import tpu as pltpu

_NEG = -1e9


def _matmul_body(x_ref, w_ref, b_ref, o_ref):
    o_ref[...] = (
        jnp.dot(x_ref[...], w_ref[...], preferred_element_type=jnp.float32)
        + b_ref[...]
    )


def _matmul(x, w, b, bn):
    m, kdim = x.shape
    n = w.shape[1]
    grid = (n // bn,)
    return pl.pallas_call(
        _matmul_body,
        grid=grid,
        in_specs=[
            pl.BlockSpec((m, kdim), lambda j: (0, 0)),
            pl.BlockSpec((kdim, bn), lambda j: (0, j)),
            pl.BlockSpec((1, bn), lambda j: (0, j)),
        ],
        out_specs=pl.BlockSpec((m, bn), lambda j: (0, j)),
        out_shape=jax.ShapeDtypeStruct((m, n), jnp.float32),
    )(x, w, b)


def _mask_body(w_ref, kcnt_ref, qi_ref, ki_ref, o_ref, *, bq, s, hi, di):
    i = pl.program_id(0)
    kcount = kcnt_ref[0]
    scores = jnp.zeros((bq, s), jnp.float32)
    for h in range(hi):
        qh = qi_ref[:, h * di:(h + 1) * di].astype(jnp.bfloat16)
        kh = ki_ref[:, h * di:(h + 1) * di].astype(jnp.bfloat16)
        dp = lax.dot_general(
            qh, kh, (((1,), (1,)), ((), ())),
            preferred_element_type=jnp.float32,
        )
        scores = scores + w_ref[h] * jnp.maximum(dp, 0.0)
    scores = jnp.where(scores == 0.0, 0.0, scores)
    bits = lax.bitcast_convert_type(scores, jnp.int32)
    keys = bits ^ ((bits >> 31) & jnp.int32(0x7FFFFFFF))

    p = jnp.full((bq, 1), np.int32(-(2 ** 31)), jnp.int32)
    for bbit in range(31, -1, -1):
        c = np.int32(-(2 ** 31)) if bbit == 31 else np.int32(1 << bbit)
        cand = p ^ c
        cnt = jnp.sum((keys >= cand).astype(jnp.int32), axis=1, keepdims=True)
        p = jnp.where(cnt >= kcount, cand, p)
    thr = p

    gt = keys > thr
    tie = keys == thr
    cnt_gt = jnp.sum(gt.astype(jnp.int32), axis=1, keepdims=True)
    m = kcount - cnt_gt

    col = lax.broadcasted_iota(jnp.int32, (bq, s), 1)
    tie_i = tie.astype(jnp.int32)
    cut = jnp.full((bq, 1), np.int32(-1), jnp.int32)
    for bbit in range(11, -1, -1):
        cand = cut + np.int32(1 << bbit)
        cnt = jnp.sum(jnp.where(col <= cand, tie_i, 0), axis=1, keepdims=True)
        cut = jnp.where(cnt <= m, cand, cut)

    member = gt | (tie & (col <= cut))
    row = i * bq + lax.broadcasted_iota(jnp.int32, (bq, s), 0)
    allowed = member & (col <= row)
    o_ref[...] = jnp.where(allowed, 0.0, _NEG).astype(jnp.float32)


def _attn_body(q_ref, k_ref, v_ref, mask_ref, o_ref, *, rsqrt_dh):
    qb = q_ref[...].astype(jnp.bfloat16)
    kb = k_ref[...].astype(jnp.bfloat16)
    logits = lax.dot_general(
        qb, kb, (((1,), (1,)), ((), ())),
        preferred_element_type=jnp.float32,
    ) / rsqrt_dh + mask_ref[...]
    mx = jnp.max(logits, axis=-1, keepdims=True)
    p = jnp.exp(logits - mx)
    denom = jnp.sum(p, axis=-1, keepdims=True)
    attn = (p / denom).astype(jnp.bfloat16)
    vb = v_ref[...].astype(jnp.bfloat16)
    o_ref[...] = jnp.dot(attn, vb, preferred_element_type=jnp.float32)


def kernel(hidden_states, Wq_idx, bq_idx, Wk_idx, bk_idx, indexer_w, Wq, Wk, Wv, Wo, k):
    B, S, D = hidden_states.shape
    Hi = indexer_w.shape[0]
    Di = Wq_idx.shape[1] // Hi
    HiDi = Hi * Di
    NH = 16
    dh = D // NH
    x = hidden_states[0]

    Wcat = jnp.concatenate([Wq_idx, Wk_idx, Wq, Wk, Wv], axis=1)
    bcat = jnp.concatenate(
        [bq_idx, bk_idx, jnp.zeros((3 * D,), jnp.float32)]
    )[None, :]
    bn = 512
    proj = _matmul(x.astype(jnp.bfloat16), Wcat.astype(jnp.bfloat16),
                   bcat, bn)

    kcount = jnp.reshape(jnp.minimum(k, 512).astype(jnp.int32), (1,))
    BQ = 256
    mask = pl.pallas_call(
        functools.partial(_mask_body, bq=BQ, s=S, hi=Hi, di=Di),
        grid=(S // BQ,),
        in_specs=[
            pl.BlockSpec(memory_space=pltpu.SMEM),
            pl.BlockSpec(memory_space=pltpu.SMEM),
            pl.BlockSpec((BQ, HiDi), lambda i: (i, 0)),
            pl.BlockSpec((S, HiDi), lambda i: (0, 1)),
        ],
        out_specs=pl.BlockSpec((BQ, S), lambda i: (i, 0)),
        out_shape=jax.ShapeDtypeStruct((S, S), jnp.float32),
    )(indexer_w, kcount, proj, proj)

    BQA = 256
    q_off = (2 * HiDi) // dh
    k_off = (2 * HiDi + D) // dh
    v_off = (2 * HiDi + 2 * D) // dh
    ctx = pl.pallas_call(
        functools.partial(_attn_body, rsqrt_dh=float(np.sqrt(dh))),
        grid=(NH, S // BQA),
        in_specs=[
            pl.BlockSpec((BQA, dh), lambda h, i: (i, q_off + h)),
            pl.BlockSpec((S, dh), lambda h, i: (0, k_off + h)),
            pl.BlockSpec((S, dh), lambda h, i: (0, v_off + h)),
            pl.BlockSpec((BQA, S), lambda h, i: (i, 0)),
        ],
        out_specs=pl.BlockSpec((BQA, dh), lambda h, i: (i, h)),
        out_shape=jax.ShapeDtypeStruct((S, D), jnp.float32),
    )(proj, proj, proj, mask)

    zerob = jnp.zeros((1, D), jnp.float32)
    out = _matmul(ctx.astype(jnp.bfloat16), Wo.astype(jnp.bfloat16), zerob, 512)
    return out[None]
